# baseline (speedup 1.0000x reference)
import functools

import jax
import jax.numpy as jnp
from jax.experimental import pallas as pl
from jax.experimental.pallas import tpu as pltpu

NUM_EXPERTS = 16
TOP_K = 2
INPUT_DIM = 2048
TILE = 1024


def _router_body(num_steps, inv_n2, x_ref, w_ref, b_ref,
                 wout_ref, iout_ref, aux_ref, acc_ref):
    i = pl.program_id(0)
    logits = jnp.dot(x_ref[...], w_ref[...],
                     preferred_element_type=jnp.float32) + b_ref[...]
    ids = jax.lax.broadcasted_iota(jnp.int32, logits.shape, 1)

    m1 = jnp.max(logits, axis=1, keepdims=True)
    i1 = jnp.min(jnp.where(logits == m1, ids, NUM_EXPERTS),
                 axis=1, keepdims=True)
    e = jnp.exp(logits - m1)
    s = jnp.sum(e, axis=1, keepdims=True)
    w1 = 1.0 / s

    masked = jnp.where(ids == i1, -jnp.inf, logits)
    m2 = jnp.max(masked, axis=1, keepdims=True)
    i2 = jnp.min(jnp.where(masked == m2, ids, NUM_EXPERTS),
                 axis=1, keepdims=True)
    w2 = jnp.exp(m2 - m1) / s

    wout_ref[...] = jnp.concatenate([w1, w2], axis=1)
    iout_ref[...] = jnp.concatenate([i1, i2], axis=1)

    probs = e / s
    psum = jnp.sum(probs, axis=0, keepdims=True)
    cnt = jnp.sum((ids == i1).astype(jnp.float32)
                  + (ids == i2).astype(jnp.float32), axis=0, keepdims=True)

    @pl.when(i == 0)
    def _():
        acc_ref[...] = jnp.zeros_like(acc_ref)

    acc_ref[0:1, :] += cnt
    acc_ref[1:2, :] += psum

    @pl.when(i == num_steps - 1)
    def _():
        aux_ref[...] = (NUM_EXPERTS * inv_n2
                        * jnp.sum(acc_ref[0:1, :] * acc_ref[1:2, :],
                                  keepdims=True))


def kernel(x, W, b):
    num_tokens = x.shape[0] * x.shape[1]
    x_flat = x.reshape(num_tokens, INPUT_DIM)
    b2 = b.reshape(1, NUM_EXPERTS)
    num_steps = num_tokens // TILE

    body = functools.partial(_router_body, num_steps,
                             1.0 / (num_tokens * num_tokens))
    weights, indices, aux = pl.pallas_call(
        body,
        grid=(num_steps,),
        in_specs=[
            pl.BlockSpec((TILE, INPUT_DIM), lambda i: (i, 0)),
            pl.BlockSpec((INPUT_DIM, NUM_EXPERTS), lambda i: (0, 0)),
            pl.BlockSpec((1, NUM_EXPERTS), lambda i: (0, 0)),
        ],
        out_specs=[
            pl.BlockSpec((TILE, TOP_K), lambda i: (i, 0)),
            pl.BlockSpec((TILE, TOP_K), lambda i: (i, 0)),
            pl.BlockSpec((1, 1), lambda i: (0, 0)),
        ],
        out_shape=[
            jax.ShapeDtypeStruct((num_tokens, TOP_K), jnp.float32),
            jax.ShapeDtypeStruct((num_tokens, TOP_K), jnp.int32),
            jax.ShapeDtypeStruct((1, 1), jnp.float32),
        ],
        scratch_shapes=[pltpu.VMEM((8, NUM_EXPERTS), jnp.float32)],
    )(x_flat, W, b2)
    return weights, indices, aux[0, 0]


# speedup vs baseline: 1.0337x; 1.0337x over previous
import functools

import jax
import jax.numpy as jnp
from jax.experimental import pallas as pl
from jax.experimental.pallas import tpu as pltpu

NUM_EXPERTS = 16
TOP_K = 2
INPUT_DIM = 2048
TILE = 2048


def _router_body(num_steps, inv_n2, x_ref, w_ref, b_ref,
                 wout_ref, iout_ref, aux_ref, acc_ref):
    i = pl.program_id(0)
    logits = jnp.dot(x_ref[...], w_ref[...],
                     preferred_element_type=jnp.float32) + b_ref[...]
    ids = jax.lax.broadcasted_iota(jnp.int32, logits.shape, 1)

    m1 = jnp.max(logits, axis=1, keepdims=True)
    i1 = jnp.min(jnp.where(logits == m1, ids, NUM_EXPERTS),
                 axis=1, keepdims=True)
    e = jnp.exp(logits - m1)
    s = jnp.sum(e, axis=1, keepdims=True)
    w1 = 1.0 / s

    masked = jnp.where(ids == i1, -jnp.inf, logits)
    m2 = jnp.max(masked, axis=1, keepdims=True)
    i2 = jnp.min(jnp.where(masked == m2, ids, NUM_EXPERTS),
                 axis=1, keepdims=True)
    w2 = jnp.exp(m2 - m1) / s

    wout_ref[...] = jnp.concatenate([w1, w2], axis=1)
    iout_ref[...] = jnp.concatenate([i1, i2], axis=1)

    probs = e / s
    psum = jnp.sum(probs, axis=0, keepdims=True)
    cnt = jnp.sum((ids == i1).astype(jnp.float32)
                  + (ids == i2).astype(jnp.float32), axis=0, keepdims=True)

    @pl.when(i == 0)
    def _():
        acc_ref[...] = jnp.zeros_like(acc_ref)

    acc_ref[0:1, :] += cnt
    acc_ref[1:2, :] += psum

    @pl.when(i == num_steps - 1)
    def _():
        aux_ref[...] = (NUM_EXPERTS * inv_n2
                        * jnp.sum(acc_ref[0:1, :] * acc_ref[1:2, :],
                                  keepdims=True))


def kernel(x, W, b):
    num_tokens = x.shape[0] * x.shape[1]
    x_flat = x.reshape(num_tokens, INPUT_DIM)
    b2 = b.reshape(1, NUM_EXPERTS)
    num_steps = num_tokens // TILE

    body = functools.partial(_router_body, num_steps,
                             1.0 / (num_tokens * num_tokens))
    weights, indices, aux = pl.pallas_call(
        body,
        grid=(num_steps,),
        in_specs=[
            pl.BlockSpec((TILE, INPUT_DIM), lambda i: (i, 0)),
            pl.BlockSpec((INPUT_DIM, NUM_EXPERTS), lambda i: (0, 0)),
            pl.BlockSpec((1, NUM_EXPERTS), lambda i: (0, 0)),
        ],
        out_specs=[
            pl.BlockSpec((TILE, TOP_K), lambda i: (i, 0)),
            pl.BlockSpec((TILE, TOP_K), lambda i: (i, 0)),
            pl.BlockSpec((1, 1), lambda i: (0, 0)),
        ],
        out_shape=[
            jax.ShapeDtypeStruct((num_tokens, TOP_K), jnp.float32),
            jax.ShapeDtypeStruct((num_tokens, TOP_K), jnp.int32),
            jax.ShapeDtypeStruct((1, 1), jnp.float32),
        ],
        scratch_shapes=[pltpu.VMEM((8, NUM_EXPERTS), jnp.float32)],
    )(x_flat, W, b2)
    return weights, indices, aux[0, 0]
